# baseline (speedup 1.0000x reference)
import functools

import jax
import jax.numpy as jnp
from jax import lax
from jax.experimental import pallas as pl
from jax.experimental.pallas import tpu as pltpu
from jax.experimental.pallas import tpu_sc as plsc

T = 2048
D = 768
F = 3072
E = 8
K = 2
RH = 128
P = T * K
TILE = 256
NUM_TILES = P // TILE + E
PAD = NUM_TILES * TILE

NW = 32
GROWS = PAD // NW
GCH = 64
GNCH = GROWS // GCH
CROWS = T // NW


def _router_body(x_ref, wr1_ref, wr2_ref, idx_ref, w_ref, xbf_ref):
    x = x_ref[...]
    xbf_ref[...] = x.astype(jnp.bfloat16)
    h = jnp.dot(x, wr1_ref[...], preferred_element_type=jnp.float32)
    h = h * jax.nn.sigmoid(h)
    logits = jnp.dot(h, wr2_ref[...], preferred_element_type=jnp.float32)
    eidx = lax.broadcasted_iota(jnp.int32, (T, E), 1)
    m1 = jnp.max(logits, axis=-1, keepdims=True)
    i1 = jnp.min(jnp.where(logits == m1, eidx, E), axis=-1, keepdims=True)
    masked = jnp.where(eidx == i1, -jnp.inf, logits)
    m2 = jnp.max(masked, axis=-1, keepdims=True)
    i2 = jnp.min(jnp.where(masked == m2, eidx, E), axis=-1, keepdims=True)
    w1 = 1.0 / (1.0 + jnp.exp(m2 - m1))
    idx_ref[...] = jnp.concatenate([i1, i2], axis=1)
    w_ref[...] = jnp.concatenate([w1, 1.0 - w1], axis=1)


def _ffn_body(texp_ref, used_ref, x_ref, w_in_ref, b_in_ref, w_out_ref,
              b_out_ref, wrow_ref, out_ref):
    g = pl.program_id(0)

    @pl.when(g < used_ref[0])
    def _():
        x = x_ref[...]
        h = jnp.dot(x, w_in_ref[0].astype(jnp.bfloat16),
                    preferred_element_type=jnp.float32)
        h = h + b_in_ref[0]
        h = h * jax.nn.sigmoid(h)
        o = jnp.dot(h.astype(jnp.bfloat16),
                    w_out_ref[0].astype(jnp.bfloat16),
                    preferred_element_type=jnp.float32)
        o = o + b_out_ref[0]
        out_ref[...] = o * wrow_ref[...]


def _gather_body(x_hbm, src_hbm, out_hbm, idx_v, rows_v, gs0, gs1, ws0, ws1):
    wid = lax.axis_index("s") * 2 + lax.axis_index("c")
    base = wid * GROWS
    gsems = [gs0, gs1]
    wsems = [ws0, ws1]

    def start_gather(j):
        slot = j % 2
        pltpu.sync_copy(src_hbm.at[pl.ds(base + j * GCH, GCH)], idx_v.at[slot])
        pltpu.async_copy(x_hbm.at[idx_v.at[slot]], rows_v.at[slot], gsems[slot])

    def wait_gather(j):
        slot = j % 2
        pltpu.make_async_copy(x_hbm.at[idx_v.at[slot]], rows_v.at[slot],
                              gsems[slot]).wait()

    def start_write(j):
        slot = j % 2
        pltpu.async_copy(rows_v.at[slot],
                         out_hbm.at[pl.ds(base + j * GCH, GCH)], wsems[slot])

    def wait_write(j):
        slot = j % 2
        pltpu.make_async_copy(rows_v.at[slot],
                              out_hbm.at[pl.ds(base + j * GCH, GCH)],
                              wsems[slot]).wait()

    start_gather(0)
    start_gather(1)
    wait_gather(0)
    start_write(0)
    wait_write(0)
    start_gather(2)
    wait_gather(1)
    start_write(1)
    wait_gather(2)
    start_write(2)
    wait_write(1)
    wait_write(2)


def _combine_body(y_hbm, p1_hbm, p2_hbm, out_hbm, i1_v, i2_v, r1_v, r2_v,
                  sem1, sem2):
    wid = lax.axis_index("s") * 2 + lax.axis_index("c")
    base = wid * CROWS
    pltpu.sync_copy(p1_hbm.at[pl.ds(base, CROWS)], i1_v)
    pltpu.sync_copy(p2_hbm.at[pl.ds(base, CROWS)], i2_v)
    c1 = pltpu.async_copy(y_hbm.at[i1_v], r1_v, sem1)
    c2 = pltpu.async_copy(y_hbm.at[i2_v], r2_v, sem2)
    c1.wait()
    c2.wait()

    @plsc.parallel_loop(0, CROWS, 1, unroll=2)
    def _add(r):
        for v in range(D // 16):
            sl = (r, pl.ds(v * 16, 16))
            r1_v[sl] = r1_v[sl] + r2_v[sl]

    pltpu.sync_copy(r1_v, out_hbm.at[pl.ds(base, CROWS)])


@functools.cache
def _sc_calls():
    mesh = plsc.VectorSubcoreMesh(core_axis_name="c", subcore_axis_name="s")
    gather_call = pl.kernel(
        _gather_body,
        out_type=jax.ShapeDtypeStruct((PAD, D // 2), jnp.int32),
        mesh=mesh,
        scratch_types=[
            pltpu.VMEM((2, GCH), jnp.int32),
            pltpu.VMEM((2, GCH, D // 2), jnp.int32),
            pltpu.SemaphoreType.DMA,
            pltpu.SemaphoreType.DMA,
            pltpu.SemaphoreType.DMA,
            pltpu.SemaphoreType.DMA,
        ],
    )
    combine_call = pl.kernel(
        _combine_body,
        out_type=jax.ShapeDtypeStruct((T, D), jnp.float32),
        mesh=mesh,
        scratch_types=[
            pltpu.VMEM((CROWS,), jnp.int32),
            pltpu.VMEM((CROWS,), jnp.int32),
            pltpu.VMEM((CROWS, D), jnp.float32),
            pltpu.VMEM((CROWS, D), jnp.float32),
            pltpu.SemaphoreType.DMA,
            pltpu.SemaphoreType.DMA,
        ],
    )
    return gather_call, combine_call


def _dispatch_meta(idx, w):
    e_flat = idx.reshape(P)
    oh = (e_flat[:, None] == jnp.arange(E, dtype=jnp.int32)[None, :])
    oh = oh.astype(jnp.int32)
    csum = jnp.cumsum(oh, axis=0)
    rank = jnp.take_along_axis(csum - oh, e_flat[:, None], axis=1)[:, 0]
    counts = csum[-1]
    padded = ((counts + TILE - 1) // TILE) * TILE
    ends = jnp.cumsum(padded)
    offs = ends - padded
    pos = offs[e_flat] + rank
    tok = jnp.arange(P, dtype=jnp.int32) // K
    src = jnp.zeros((PAD,), jnp.int32).at[pos].set(tok)
    wrow = jnp.zeros((PAD,), jnp.float32).at[pos].set(w.reshape(P))
    tstart = jnp.arange(NUM_TILES, dtype=jnp.int32) * TILE
    texp = jnp.searchsorted(offs, tstart, side="right").astype(jnp.int32) - 1
    used = (ends[-1] // TILE).astype(jnp.int32)[None]
    nrows = ends[-1].astype(jnp.int32)[None]
    return src, wrow, pos, texp, used, nrows


def kernel(hidden_states, wr1, wr2, w_in, b_in, w_out, b_out):
    Bz, Sz, Dm = hidden_states.shape
    x = hidden_states.reshape(T, D)

    idx, w, xbf = pl.pallas_call(
        _router_body,
        out_shape=[
            jax.ShapeDtypeStruct((T, K), jnp.int32),
            jax.ShapeDtypeStruct((T, K), jnp.float32),
            jax.ShapeDtypeStruct((T, D), jnp.bfloat16),
        ],
    )(x, wr1, wr2)

    src, wrow, pos, texp, used, nrows = _dispatch_meta(idx, w)

    gather_call, combine_call = _sc_calls()
    xi = lax.bitcast_convert_type(xbf.reshape(T, D // 2, 2), jnp.int32)
    xgi = gather_call(xi, src)
    xg = lax.bitcast_convert_type(xgi, jnp.bfloat16).reshape(PAD, D)

    grid_spec = pltpu.PrefetchScalarGridSpec(
        num_scalar_prefetch=2,
        grid=(NUM_TILES,),
        in_specs=[
            pl.BlockSpec((TILE, D), lambda g, tr, ur: (g, 0)),
            pl.BlockSpec((1, D, F), lambda g, tr, ur: (tr[g], 0, 0)),
            pl.BlockSpec((1, 1, F), lambda g, tr, ur: (tr[g], 0, 0)),
            pl.BlockSpec((1, F, D), lambda g, tr, ur: (tr[g], 0, 0)),
            pl.BlockSpec((1, 1, D), lambda g, tr, ur: (tr[g], 0, 0)),
            pl.BlockSpec((TILE, 1), lambda g, tr, ur: (g, 0)),
        ],
        out_specs=pl.BlockSpec((TILE, D), lambda g, tr, ur: (g, 0)),
    )
    y = pl.pallas_call(
        _ffn_body,
        grid_spec=grid_spec,
        out_shape=jax.ShapeDtypeStruct((PAD, D), jnp.float32),
        compiler_params=pltpu.CompilerParams(
            dimension_semantics=("arbitrary",),
        ),
    )(texp, used, xg, w_in, b_in.reshape(E, 1, F), w_out,
      b_out.reshape(E, 1, D), wrow.reshape(PAD, 1))

    p1 = pos.reshape(T, K)[:, 0]
    p2 = pos.reshape(T, K)[:, 1]
    out = combine_call(y, p1, p2)
    return out.reshape(Bz, Sz, Dm)


# speedup vs baseline: 1.5831x; 1.5831x over previous
import functools

import jax
import jax.numpy as jnp
from jax import lax
from jax.experimental import pallas as pl
from jax.experimental.pallas import tpu as pltpu
from jax.experimental.pallas import tpu_sc as plsc

T = 2048
D = 768
F = 3072
E = 8
K = 2
RH = 128
P = T * K
TILE = 128
NUM_TILES = P // TILE + E
PAD = NUM_TILES * TILE

NW = 32
GROWS = PAD // NW
GCH = GROWS // 2
CROWS = T // NW


def _router_body(x_ref, wr1_ref, wr2_ref, idx_ref, w_ref):
    x = x_ref[...]
    h = jnp.dot(x, wr1_ref[...], preferred_element_type=jnp.float32)
    h = h * jax.nn.sigmoid(h)
    logits = jnp.dot(h, wr2_ref[...], preferred_element_type=jnp.float32)
    eidx = lax.broadcasted_iota(jnp.int32, (T, E), 1)
    m1 = jnp.max(logits, axis=-1, keepdims=True)
    i1 = jnp.min(jnp.where(logits == m1, eidx, E), axis=-1, keepdims=True)
    masked = jnp.where(eidx == i1, -jnp.inf, logits)
    m2 = jnp.max(masked, axis=-1, keepdims=True)
    i2 = jnp.min(jnp.where(masked == m2, eidx, E), axis=-1, keepdims=True)
    w1 = 1.0 / (1.0 + jnp.exp(m2 - m1))
    idx_ref[...] = jnp.concatenate([i1, i2], axis=1)
    w_ref[...] = jnp.concatenate([w1, 1.0 - w1], axis=1)


def _ffn_body(texp_ref, used_ref, x_ref, w_in_ref, b_in_ref, w_out_ref,
              b_out_ref, wrow_ref, out_ref):
    g = pl.program_id(0)

    @pl.when(g < used_ref[0])
    def _():
        x = x_ref[...].astype(jnp.bfloat16)
        h = jnp.dot(x, w_in_ref[0].astype(jnp.bfloat16),
                    preferred_element_type=jnp.float32)
        h = h + b_in_ref[0]
        h = h * jax.nn.sigmoid(h)
        o = jnp.dot(h.astype(jnp.bfloat16),
                    w_out_ref[0].astype(jnp.bfloat16),
                    preferred_element_type=jnp.float32)
        o = o + b_out_ref[0]
        out_ref[...] = o * wrow_ref[...]


def _gather_body(x_hbm, src_hbm, out_hbm, idx_v, rows_v, gs0, gs1, ws0, ws1):
    wid = lax.axis_index("s") * 2 + lax.axis_index("c")
    base = wid * GROWS
    gsems = [gs0, gs1]
    wsems = [ws0, ws1]

    def start_gather(j):
        pltpu.sync_copy(src_hbm.at[pl.ds(base + j * GCH, GCH)], idx_v.at[j])
        pltpu.async_copy(x_hbm.at[idx_v.at[j]], rows_v.at[j], gsems[j])

    def wait_gather(j):
        pltpu.make_async_copy(x_hbm.at[idx_v.at[j]], rows_v.at[j],
                              gsems[j]).wait()

    def start_write(j):
        pltpu.async_copy(rows_v.at[j],
                         out_hbm.at[pl.ds(base + j * GCH, GCH)], wsems[j])

    def wait_write(j):
        pltpu.make_async_copy(rows_v.at[j],
                              out_hbm.at[pl.ds(base + j * GCH, GCH)],
                              wsems[j]).wait()

    start_gather(0)
    start_gather(1)
    wait_gather(0)
    start_write(0)
    wait_gather(1)
    start_write(1)
    wait_write(0)
    wait_write(1)


def _combine_body(y_hbm, p1_hbm, p2_hbm, out_hbm, i1_v, i2_v, r1_v, r2_v,
                  sem1, sem2):
    wid = lax.axis_index("s") * 2 + lax.axis_index("c")
    base = wid * CROWS
    pltpu.sync_copy(p1_hbm.at[pl.ds(base, CROWS)], i1_v)
    pltpu.sync_copy(p2_hbm.at[pl.ds(base, CROWS)], i2_v)
    c1 = pltpu.async_copy(y_hbm.at[i1_v], r1_v, sem1)
    c2 = pltpu.async_copy(y_hbm.at[i2_v], r2_v, sem2)
    c1.wait()
    c2.wait()

    @plsc.parallel_loop(0, CROWS, 1, unroll=2)
    def _add(r):
        for v in range(D // 16):
            sl = (r, pl.ds(v * 16, 16))
            r1_v[sl] = r1_v[sl] + r2_v[sl]

    pltpu.sync_copy(r1_v, out_hbm.at[pl.ds(base, CROWS)])


@functools.cache
def _sc_calls():
    mesh = plsc.VectorSubcoreMesh(core_axis_name="c", subcore_axis_name="s")
    gather_call = pl.kernel(
        _gather_body,
        out_type=jax.ShapeDtypeStruct((PAD, D), jnp.float32),
        mesh=mesh,
        scratch_types=[
            pltpu.VMEM((2, GCH), jnp.int32),
            pltpu.VMEM((2, GCH, D), jnp.float32),
            pltpu.SemaphoreType.DMA,
            pltpu.SemaphoreType.DMA,
            pltpu.SemaphoreType.DMA,
            pltpu.SemaphoreType.DMA,
        ],
    )
    combine_call = pl.kernel(
        _combine_body,
        out_type=jax.ShapeDtypeStruct((T, D), jnp.float32),
        mesh=mesh,
        scratch_types=[
            pltpu.VMEM((CROWS,), jnp.int32),
            pltpu.VMEM((CROWS,), jnp.int32),
            pltpu.VMEM((CROWS, D), jnp.float32),
            pltpu.VMEM((CROWS, D), jnp.float32),
            pltpu.SemaphoreType.DMA,
            pltpu.SemaphoreType.DMA,
        ],
    )
    return gather_call, combine_call


def _dispatch_meta(idx, w):
    e_flat = idx.reshape(P)
    oh = (e_flat[:, None] == jnp.arange(E, dtype=jnp.int32)[None, :])
    oh = oh.astype(jnp.int32)
    csum = jnp.cumsum(oh, axis=0)
    rank = jnp.take_along_axis(csum - oh, e_flat[:, None], axis=1)[:, 0]
    counts = csum[-1]
    padded = ((counts + TILE - 1) // TILE) * TILE
    ends = jnp.cumsum(padded)
    offs = ends - padded
    pos = offs[e_flat] + rank
    tok = jnp.arange(P, dtype=jnp.int32) // K
    src = jnp.zeros((PAD,), jnp.int32).at[pos].set(tok)
    wrow = jnp.zeros((PAD,), jnp.float32).at[pos].set(w.reshape(P))
    tstart = jnp.arange(NUM_TILES, dtype=jnp.int32) * TILE
    texp = jnp.searchsorted(offs, tstart, side="right").astype(jnp.int32) - 1
    used = (ends[-1] // TILE).astype(jnp.int32)[None]
    nrows = ends[-1].astype(jnp.int32)[None]
    return src, wrow, pos, texp, used, nrows


def kernel(hidden_states, wr1, wr2, w_in, b_in, w_out, b_out):
    Bz, Sz, Dm = hidden_states.shape
    x = hidden_states.reshape(T, D)

    idx, w = pl.pallas_call(
        _router_body,
        out_shape=[
            jax.ShapeDtypeStruct((T, K), jnp.int32),
            jax.ShapeDtypeStruct((T, K), jnp.float32),
        ],
    )(x, wr1, wr2)

    src, wrow, pos, texp, used, nrows = _dispatch_meta(idx, w)

    gather_call, combine_call = _sc_calls()
    xg = gather_call(x, src)

    grid_spec = pltpu.PrefetchScalarGridSpec(
        num_scalar_prefetch=2,
        grid=(NUM_TILES,),
        in_specs=[
            pl.BlockSpec((TILE, D), lambda g, tr, ur: (g, 0)),
            pl.BlockSpec((1, D, F), lambda g, tr, ur: (tr[g], 0, 0)),
            pl.BlockSpec((1, 1, F), lambda g, tr, ur: (tr[g], 0, 0)),
            pl.BlockSpec((1, F, D), lambda g, tr, ur: (tr[g], 0, 0)),
            pl.BlockSpec((1, 1, D), lambda g, tr, ur: (tr[g], 0, 0)),
            pl.BlockSpec((TILE, 1), lambda g, tr, ur: (g, 0)),
        ],
        out_specs=pl.BlockSpec((TILE, D), lambda g, tr, ur: (g, 0)),
    )
    y = pl.pallas_call(
        _ffn_body,
        grid_spec=grid_spec,
        out_shape=jax.ShapeDtypeStruct((PAD, D), jnp.float32),
        compiler_params=pltpu.CompilerParams(
            dimension_semantics=("arbitrary",),
        ),
    )(texp, used, xg, w_in, b_in.reshape(E, 1, F), w_out,
      b_out.reshape(E, 1, D), wrow.reshape(PAD, 1))

    p1 = pos.reshape(T, K)[:, 0]
    p2 = pos.reshape(T, K)[:, 1]
    out = combine_call(y, p1, p2)
    return out.reshape(Bz, Sz, Dm)
